# Initial kernel scaffold; baseline (speedup 1.0000x reference)
import functools

import jax
import jax.numpy as jnp
from jax import lax
from jax.experimental import pallas as pl
from jax.experimental.pallas import tpu as pltpu
from jax.experimental.pallas import tpu_sc as plsc

N = 10000
E = 320000
HID = 128
CLS = 4
NLAYER = 5

NC = 2
NS = 16
NW = NC * NS
EP = E // NW
ECH = 80
NCH = EP // ECH
RPT = N // NS
ZR = 125


def _make_agg(h):
  mesh = plsc.VectorSubcoreMesh(core_axis_name="c", subcore_axis_name="s")

  @functools.partial(
      pl.kernel,
      mesh=mesh,
      out_type=jax.ShapeDtypeStruct((NC, N, h), jnp.float32),
      scratch_types=[
          pltpu.VMEM((ECH,), jnp.int32),
          pltpu.VMEM((ECH,), jnp.int32),
          pltpu.VMEM((ECH, h), jnp.float32),
          pltpu.VMEM((ZR, h), jnp.float32),
          pltpu.VMEM_SHARED((N, h), jnp.float32),
          pltpu.SemaphoreType.DMA,
      ],
  )
  def agg(rows_hbm, src_hbm, dst_hbm, out_hbm, src_v, dst_v, buf_v, zero_v,
          acc_sh, sem):
    cid = lax.axis_index("c")
    sid = lax.axis_index("s")
    wid = sid * NC + cid

    def zero_row(i, carry):
      for j in range(h // 16):
        zero_v[i, pl.ds(j * 16, 16)] = jnp.zeros((16,), jnp.float32)
      return carry

    lax.fori_loop(0, ZR, zero_row, 0)
    for r in range(RPT // ZR):
      pltpu.sync_copy(zero_v, acc_sh.at[pl.ds(sid * RPT + r * ZR, ZR)])
    plsc.subcore_barrier()

    base = wid * EP

    def chunk(k, carry):
      off = base + k * ECH
      pltpu.sync_copy(src_hbm.at[pl.ds(off, ECH)], src_v)
      pltpu.sync_copy(dst_hbm.at[pl.ds(off, ECH)], dst_v)
      pltpu.async_copy(rows_hbm.at[src_v], buf_v, sem).wait()
      pltpu.sync_copy(buf_v, acc_sh.at[dst_v], add=True)
      return carry

    lax.fori_loop(0, NCH, chunk, 0)

    plsc.subcore_barrier()
    pltpu.sync_copy(acc_sh.at[pl.ds(sid * RPT, RPT)],
                    out_hbm.at[cid, pl.ds(sid * RPT, RPT)])

  return agg


_agg128 = _make_agg(HID)
_agg16 = _make_agg(16)

_TC_R = 1000
_GRID = N // _TC_R


def _row_spec(w=HID):
  return pl.BlockSpec((_TC_R, w), lambda i: (i, 0))


def _fixed_spec(a, b):
  return pl.BlockSpec((a, b), lambda i: (0, 0))


def _dis_body(c0_ref, c1_ref, d2_ref, d1_ref):
  cnt16 = c0_ref[...] + c1_ref[...]
  cnt = jnp.broadcast_to(cnt16[:, :1], (_TC_R, HID))
  d2_ref[...] = lax.rsqrt(cnt + 2.0)
  d1_ref[...] = lax.rsqrt(cnt + 1.0)


_dis_call = pl.pallas_call(
    _dis_body,
    grid=(_GRID,),
    in_specs=[_row_spec(16), _row_spec(16)],
    out_specs=[_row_spec(), _row_spec()],
    out_shape=[jax.ShapeDtypeStruct((N, HID), jnp.float32)] * 2,
)


def _in_body(x_ref, w_ref, d2_ref, hp_ref):
  hp_ref[...] = d2_ref[...] * jnp.dot(
      x_ref[...], w_ref[...], preferred_element_type=jnp.float32)


_in_call = pl.pallas_call(
    _in_body,
    grid=(_GRID,),
    in_specs=[_row_spec(), _fixed_spec(HID, HID), _row_spec()],
    out_specs=_row_spec(),
    out_shape=jax.ShapeDtypeStruct((N, HID), jnp.float32),
)


def _mid_body(p0_ref, p1_ref, hp_ref, d2_ref, b_ref, w_ref, o_ref):
  g = d2_ref[...] * (p0_ref[...] + p1_ref[...] + 2.0 * hp_ref[...]) + b_ref[...]
  g = jnp.maximum(g, 0.0)
  o_ref[...] = d2_ref[...] * jnp.dot(
      g, w_ref[...], preferred_element_type=jnp.float32)


_mid_call = pl.pallas_call(
    _mid_body,
    grid=(_GRID,),
    in_specs=[_row_spec(), _row_spec(), _row_spec(), _row_spec(),
              _fixed_spec(1, HID), _fixed_spec(HID, HID)],
    out_specs=_row_spec(),
    out_shape=jax.ShapeDtypeStruct((N, HID), jnp.float32),
)


def _pre_body(p0_ref, p1_ref, hp_ref, d2_ref, d1_ref, b_ref, o_ref):
  g = d2_ref[...] * (p0_ref[...] + p1_ref[...] + 2.0 * hp_ref[...]) + b_ref[...]
  o_ref[...] = d1_ref[...] * jnp.maximum(g, 0.0)


_pre_call = pl.pallas_call(
    _pre_body,
    grid=(_GRID,),
    in_specs=[_row_spec(), _row_spec(), _row_spec(), _row_spec(), _row_spec(),
              _fixed_spec(1, HID)],
    out_specs=_row_spec(),
    out_shape=jax.ShapeDtypeStruct((N, HID), jnp.float32),
)


def _out_body(q0_ref, q1_ref, hp_ref, d1_ref, w_ref, b_ref, o_ref):
  agg = d1_ref[...] * (q0_ref[...] + q1_ref[...] + hp_ref[...])
  o_ref[...] = jnp.dot(
      agg, w_ref[...], preferred_element_type=jnp.float32) + b_ref[...]


_out_call = pl.pallas_call(
    _out_body,
    grid=(_GRID,),
    in_specs=[_row_spec(), _row_spec(), _row_spec(), _row_spec(),
              _fixed_spec(HID, HID), _fixed_spec(1, HID)],
    out_specs=_row_spec(),
    out_shape=jax.ShapeDtypeStruct((N, HID), jnp.float32),
)


def kernel(x, edge_index, W_in, b_in, W_h, b_h, W_out, b_out):
  src = edge_index[0]
  dst = edge_index[1]

  ones16 = jnp.ones((N, 16), jnp.float32)
  cnt = _agg16(ones16, src, dst)
  d2m, d1m = _dis_call(cnt[0], cnt[1])

  hp = _in_call(x, W_in, d2m)
  biases = [b_in] + [b_h[i] for i in range(NLAYER - 1)]
  for j in range(NLAYER):
    p = _agg128(hp, src, dst)
    hp = _mid_call(p[0], p[1], hp, d2m, biases[j].reshape(1, HID), W_h[j])

  p = _agg128(hp, src, dst)
  hp6 = _pre_call(p[0], p[1], hp, d2m, d1m, b_h[NLAYER - 1].reshape(1, HID))

  q = _agg128(hp6, src, dst)
  wo = jnp.zeros((HID, HID), jnp.float32).at[:, :CLS].set(W_out)
  bo = jnp.zeros((1, HID), jnp.float32).at[0, :CLS].set(b_out)
  out128 = _out_call(q[0], q[1], hp6, d1m, wo, bo)
  return out128[:, :CLS]


# speedup vs baseline: 6.9571x; 6.9571x over previous
import functools

import jax
import jax.numpy as jnp
from jax import lax
from jax.experimental import pallas as pl
from jax.experimental.pallas import tpu as pltpu
from jax.experimental.pallas import tpu_sc as plsc

N = 10000
NPAD = 10240
E = 320000
HID = 128
CLS = 4
NLAYER = 5

NC = 2
NS = 16
NW = NC * NS
EP = E // NW
ECH = 80
NCH = EP // ECH
RPT = NPAD // NS
ZR = 128


def _make_agg(h):
  mesh = plsc.VectorSubcoreMesh(core_axis_name="c", subcore_axis_name="s")

  @functools.partial(
      pl.kernel,
      mesh=mesh,
      out_type=jax.ShapeDtypeStruct((NC, NPAD, h), jnp.float32),
      scratch_types=[
          pltpu.VMEM((ECH,), jnp.int32),
          pltpu.VMEM((ECH,), jnp.int32),
          pltpu.VMEM((ECH, h), jnp.float32),
          pltpu.VMEM((ZR, h), jnp.float32),
          pltpu.VMEM_SHARED((NPAD, h), jnp.float32),
          pltpu.SemaphoreType.DMA,
      ],
  )
  def agg(rows_hbm, src_hbm, dst_hbm, out_hbm, src_v, dst_v, buf_v, zero_v,
          acc_sh, sem):
    cid = lax.axis_index("c")
    sid = lax.axis_index("s")
    wid = sid * NC + cid

    def zero_row(i, carry):
      for j in range(h // 16):
        zero_v[i, pl.ds(j * 16, 16)] = jnp.zeros((16,), jnp.float32)
      return carry

    lax.fori_loop(0, ZR, zero_row, 0)
    for r in range(RPT // ZR):
      pltpu.sync_copy(zero_v, acc_sh.at[pl.ds(sid * RPT + r * ZR, ZR)])
    plsc.subcore_barrier()

    base = wid * EP

    def chunk(k, carry):
      off = base + k * ECH
      pltpu.sync_copy(src_hbm.at[pl.ds(off, ECH)], src_v)
      pltpu.sync_copy(dst_hbm.at[pl.ds(off, ECH)], dst_v)
      pltpu.async_copy(rows_hbm.at[src_v], buf_v, sem).wait()
      pltpu.sync_copy(buf_v, acc_sh.at[dst_v], add=True)
      return carry

    lax.fori_loop(0, NCH, chunk, 0)

    plsc.subcore_barrier()
    pltpu.sync_copy(acc_sh.at[pl.ds(sid * RPT, RPT)],
                    out_hbm.at[cid, pl.ds(sid * RPT, RPT)])

  return agg


@functools.lru_cache(maxsize=None)
def _get_agg(h):
  return _make_agg(h)

_TC_R = 1000
_GRID = N // _TC_R


def _row_spec(w=HID):
  return pl.BlockSpec((_TC_R, w), lambda i: (i, 0))


def _fixed_spec(a, b):
  return pl.BlockSpec((a, b), lambda i: (0, 0))


def _dis_body(c0_ref, c1_ref, d2_ref, d1_ref):
  cnt = c0_ref[...] + c1_ref[...]
  d2_ref[...] = lax.rsqrt(cnt + 2.0)
  d1_ref[...] = lax.rsqrt(cnt + 1.0)


_dis_call = pl.pallas_call(
    _dis_body,
    grid=(_GRID,),
    in_specs=[_row_spec(), _row_spec()],
    out_specs=[_row_spec(), _row_spec()],
    out_shape=[jax.ShapeDtypeStruct((N, HID), jnp.float32)] * 2,
)


def _in_body(x_ref, w_ref, d2_ref, hp_ref):
  hp_ref[...] = d2_ref[...] * jnp.dot(
      x_ref[...], w_ref[...], preferred_element_type=jnp.float32)


_in_call = pl.pallas_call(
    _in_body,
    grid=(_GRID,),
    in_specs=[_row_spec(), _fixed_spec(HID, HID), _row_spec()],
    out_specs=_row_spec(),
    out_shape=jax.ShapeDtypeStruct((N, HID), jnp.float32),
)


def _mid_body(p0_ref, p1_ref, hp_ref, d2_ref, b_ref, w_ref, o_ref):
  g = d2_ref[...] * (p0_ref[...] + p1_ref[...] + 2.0 * hp_ref[...]) + b_ref[...]
  g = jnp.maximum(g, 0.0)
  o_ref[...] = d2_ref[...] * jnp.dot(
      g, w_ref[...], preferred_element_type=jnp.float32)


_mid_call = pl.pallas_call(
    _mid_body,
    grid=(_GRID,),
    in_specs=[_row_spec(), _row_spec(), _row_spec(), _row_spec(),
              _fixed_spec(1, HID), _fixed_spec(HID, HID)],
    out_specs=_row_spec(),
    out_shape=jax.ShapeDtypeStruct((N, HID), jnp.float32),
)


def _pre_body(p0_ref, p1_ref, hp_ref, d2_ref, d1_ref, b_ref, o_ref):
  g = d2_ref[...] * (p0_ref[...] + p1_ref[...] + 2.0 * hp_ref[...]) + b_ref[...]
  o_ref[...] = d1_ref[...] * jnp.maximum(g, 0.0)


_pre_call = pl.pallas_call(
    _pre_body,
    grid=(_GRID,),
    in_specs=[_row_spec(), _row_spec(), _row_spec(), _row_spec(), _row_spec(),
              _fixed_spec(1, HID)],
    out_specs=_row_spec(),
    out_shape=jax.ShapeDtypeStruct((N, HID), jnp.float32),
)


def _out_body(q0_ref, q1_ref, hp_ref, d1_ref, w_ref, b_ref, o_ref):
  agg = d1_ref[...] * (q0_ref[...] + q1_ref[...] + hp_ref[...])
  o_ref[...] = jnp.dot(
      agg, w_ref[...], preferred_element_type=jnp.float32) + b_ref[...]


_out_call = pl.pallas_call(
    _out_body,
    grid=(_GRID,),
    in_specs=[_row_spec(), _row_spec(), _row_spec(), _row_spec(),
              _fixed_spec(HID, HID), _fixed_spec(1, HID)],
    out_specs=_row_spec(),
    out_shape=jax.ShapeDtypeStruct((N, HID), jnp.float32),
)


def kernel(x, edge_index, W_in, b_in, W_h, b_h, W_out, b_out):
  src = edge_index[0]
  dst = edge_index[1]

  agg128 = _get_agg(HID)

  ones128 = jnp.ones((N, HID), jnp.float32)
  cnt = agg128(ones128, src, dst)
  d2m, d1m = _dis_call(cnt[0], cnt[1])

  hp = _in_call(x, W_in, d2m)
  biases = [b_in] + [b_h[i] for i in range(NLAYER - 1)]
  for j in range(NLAYER):
    p = agg128(hp, src, dst)
    hp = _mid_call(p[0], p[1], hp, d2m, biases[j].reshape(1, HID), W_h[j])

  p = agg128(hp, src, dst)
  hp6 = _pre_call(p[0], p[1], hp, d2m, d1m, b_h[NLAYER - 1].reshape(1, HID))

  q = agg128(hp6, src, dst)
  wo = jnp.zeros((HID, HID), jnp.float32).at[:, :CLS].set(W_out)
  bo = jnp.zeros((1, HID), jnp.float32).at[0, :CLS].set(b_out)
  out128 = _out_call(q[0], q[1], hp6, d1m, wo, bo)
  return out128[:, :CLS]


# speedup vs baseline: 14.1134x; 2.0286x over previous
import functools

import jax
import jax.numpy as jnp
from jax import lax
from jax.experimental import pallas as pl
from jax.experimental.pallas import tpu as pltpu
from jax.experimental.pallas import tpu_sc as plsc

N = 10000
NPAD = 10240
E = 320000
HID = 128
CLS = 4
NLAYER = 5
CW = 16

NC = 2
NS = 16
NW = NC * NS
EP = E // NW
ECH = 80
NCH = EP // ECH
RPT = NPAD // NS
ZR = 128


def _make_agg():
  mesh = plsc.VectorSubcoreMesh(core_axis_name="c", subcore_axis_name="s")

  @functools.partial(
      pl.kernel,
      mesh=mesh,
      out_type=jax.ShapeDtypeStruct((NC, NPAD, HID), jnp.float32),
      scratch_types=[
          pltpu.VMEM((EP,), jnp.int32),
          pltpu.VMEM((NCH, ECH), jnp.int32),
          pltpu.VMEM((ECH, HID), jnp.float32),
          pltpu.VMEM((ECH, HID), jnp.float32),
          pltpu.VMEM_SHARED((NPAD, HID), jnp.float32),
          pltpu.SemaphoreType.DMA,
          pltpu.SemaphoreType.DMA,
      ],
  )
  def agg(rows_hbm, src_hbm, dst_hbm, out_hbm, src_v, dst_v, buf0, buf1,
          acc_sh, sem0, sem1):
    cid = lax.axis_index("c")
    sid = lax.axis_index("s")
    wid = sid * NC + cid

    def zero_row(i, carry):
      for j in range(HID // 16):
        buf0[i, pl.ds(j * 16, 16)] = jnp.zeros((16,), jnp.float32)
      return carry

    lax.fori_loop(0, ECH, zero_row, 0)
    for r in range(RPT // ECH):
      pltpu.sync_copy(buf0, acc_sh.at[pl.ds(sid * RPT + r * ECH, ECH)])
    pltpu.sync_copy(src_hbm.at[pl.ds(wid * EP, EP)], src_v)
    pltpu.sync_copy(dst_hbm.at[wid], dst_v)
    plsc.subcore_barrier()

    pltpu.async_copy(rows_hbm.at[src_v.at[pl.ds(0, ECH)]], buf0, sem0)

    def pair(kk, carry):
      k0 = 2 * kk
      pltpu.make_async_copy(rows_hbm.at[src_v.at[pl.ds(k0 * ECH, ECH)]], buf0, sem0).wait()
      pltpu.async_copy(rows_hbm.at[src_v.at[pl.ds((k0 + 1) * ECH, ECH)]], buf1, sem1)
      pltpu.sync_copy(buf0, acc_sh.at[dst_v.at[k0]], add=True)
      pltpu.make_async_copy(rows_hbm.at[src_v.at[pl.ds((k0 + 1) * ECH, ECH)]], buf1, sem1).wait()

      @pl.when(k0 + 2 < NCH)
      def _():
        pltpu.async_copy(rows_hbm.at[src_v.at[pl.ds((k0 + 2) * ECH, ECH)]], buf0, sem0)

      pltpu.sync_copy(buf1, acc_sh.at[dst_v.at[k0 + 1]], add=True)
      return carry

    lax.fori_loop(0, (NCH - 1) // 2, pair, 0)
    pltpu.make_async_copy(rows_hbm.at[src_v.at[pl.ds((NCH - 1) * ECH, ECH)]], buf0, sem0).wait()
    pltpu.sync_copy(buf0, acc_sh.at[dst_v.at[NCH - 1]], add=True)

    plsc.subcore_barrier()
    pltpu.sync_copy(acc_sh.at[pl.ds(sid * RPT, RPT)],
                    out_hbm.at[cid, pl.ds(sid * RPT, RPT)])

  return agg


def _make_cnt():
  mesh = plsc.VectorSubcoreMesh(core_axis_name="c", subcore_axis_name="s")

  @functools.partial(
      pl.kernel,
      mesh=mesh,
      out_type=jax.ShapeDtypeStruct((NC, NPAD, CW), jnp.float32),
      scratch_types=[
          pltpu.VMEM((NCH, ECH), jnp.int32),
          pltpu.VMEM((ECH, CW), jnp.float32),
          pltpu.VMEM((ZR, CW), jnp.float32),
          pltpu.VMEM_SHARED((NPAD, CW), jnp.float32),
      ],
  )
  def cnt(dst_hbm, out_hbm, dst_v, one_v, zero_v, acc_sh):
    cid = lax.axis_index("c")
    sid = lax.axis_index("s")
    wid = sid * NC + cid

    def fill_one(i, carry):
      one_v[i, pl.ds(0, 16)] = jnp.full((16,), 1.0, jnp.float32)
      return carry

    lax.fori_loop(0, ECH, fill_one, 0)

    def fill_zero(i, carry):
      zero_v[i, pl.ds(0, 16)] = jnp.zeros((16,), jnp.float32)
      return carry

    lax.fori_loop(0, ZR, fill_zero, 0)
    for r in range(RPT // ZR):
      pltpu.sync_copy(zero_v, acc_sh.at[pl.ds(sid * RPT + r * ZR, ZR)])
    pltpu.sync_copy(dst_hbm.at[wid], dst_v)
    plsc.subcore_barrier()

    def chunk(k, carry):
      pltpu.sync_copy(one_v, acc_sh.at[dst_v.at[k]], add=True)
      return carry

    lax.fori_loop(0, NCH, chunk, 0)

    plsc.subcore_barrier()
    pltpu.sync_copy(acc_sh.at[pl.ds(sid * RPT, RPT)],
                    out_hbm.at[cid, pl.ds(sid * RPT, RPT)])

  return cnt


@functools.lru_cache(maxsize=None)
def _get_agg():
  return _make_agg()


@functools.lru_cache(maxsize=None)
def _get_cnt():
  return _make_cnt()


_TC_R = 1000
_GRID = N // _TC_R


def _row_spec(w=HID):
  return pl.BlockSpec((_TC_R, w), lambda i: (i, 0))


def _fixed_spec(a, b):
  return pl.BlockSpec((a, b), lambda i: (0, 0))


def _dis_body(c0_ref, c1_ref, d2_ref, d1_ref):
  cnt16 = c0_ref[...] + c1_ref[...]
  cnt = jnp.broadcast_to(cnt16[:, :1], (_TC_R, HID))
  d2_ref[...] = lax.rsqrt(cnt + 2.0)
  d1_ref[...] = lax.rsqrt(cnt + 1.0)


_dis_call = pl.pallas_call(
    _dis_body,
    grid=(_GRID,),
    in_specs=[_row_spec(CW), _row_spec(CW)],
    out_specs=[_row_spec(), _row_spec()],
    out_shape=[jax.ShapeDtypeStruct((N, HID), jnp.float32)] * 2,
)


def _in_body(x_ref, w_ref, d2_ref, hp_ref):
  hp_ref[...] = d2_ref[...] * jnp.dot(
      x_ref[...], w_ref[...], preferred_element_type=jnp.float32)


_in_call = pl.pallas_call(
    _in_body,
    grid=(_GRID,),
    in_specs=[_row_spec(), _fixed_spec(HID, HID), _row_spec()],
    out_specs=_row_spec(),
    out_shape=jax.ShapeDtypeStruct((N, HID), jnp.float32),
)


def _mid_body(p0_ref, p1_ref, hp_ref, d2_ref, b_ref, w_ref, o_ref):
  g = d2_ref[...] * (p0_ref[...] + p1_ref[...] + 2.0 * hp_ref[...]) + b_ref[...]
  g = jnp.maximum(g, 0.0)
  o_ref[...] = d2_ref[...] * jnp.dot(
      g, w_ref[...], preferred_element_type=jnp.float32)


_mid_call = pl.pallas_call(
    _mid_body,
    grid=(_GRID,),
    in_specs=[_row_spec(), _row_spec(), _row_spec(), _row_spec(),
              _fixed_spec(1, HID), _fixed_spec(HID, HID)],
    out_specs=_row_spec(),
    out_shape=jax.ShapeDtypeStruct((N, HID), jnp.float32),
)


def _pre_body(p0_ref, p1_ref, hp_ref, d2_ref, d1_ref, b_ref, o_ref):
  g = d2_ref[...] * (p0_ref[...] + p1_ref[...] + 2.0 * hp_ref[...]) + b_ref[...]
  o_ref[...] = d1_ref[...] * jnp.maximum(g, 0.0)


_pre_call = pl.pallas_call(
    _pre_body,
    grid=(_GRID,),
    in_specs=[_row_spec(), _row_spec(), _row_spec(), _row_spec(), _row_spec(),
              _fixed_spec(1, HID)],
    out_specs=_row_spec(),
    out_shape=jax.ShapeDtypeStruct((N, HID), jnp.float32),
)


def _out_body(q0_ref, q1_ref, hp_ref, d1_ref, w_ref, b_ref, o_ref):
  agg = d1_ref[...] * (q0_ref[...] + q1_ref[...] + hp_ref[...])
  o_ref[...] = jnp.dot(
      agg, w_ref[...], preferred_element_type=jnp.float32) + b_ref[...]


_out_call = pl.pallas_call(
    _out_body,
    grid=(_GRID,),
    in_specs=[_row_spec(), _row_spec(), _row_spec(), _row_spec(),
              _fixed_spec(HID, HID), _fixed_spec(1, HID)],
    out_specs=_row_spec(),
    out_shape=jax.ShapeDtypeStruct((N, HID), jnp.float32),
)


def kernel(x, edge_index, W_in, b_in, W_h, b_h, W_out, b_out):
  src1 = edge_index[0]
  dst3 = edge_index[1].reshape(NW, NCH, ECH)

  agg = _get_agg()
  cnt = _get_cnt()(dst3)
  d2m, d1m = _dis_call(cnt[0, :, :], cnt[1, :, :])

  hp = _in_call(x, W_in, d2m)
  biases = [b_in] + [b_h[i] for i in range(NLAYER - 1)]
  for j in range(NLAYER):
    p = agg(hp, src1, dst3)
    hp = _mid_call(p[0], p[1], hp, d2m, biases[j].reshape(1, HID), W_h[j])

  p = agg(hp, src1, dst3)
  hp6 = _pre_call(p[0], p[1], hp, d2m, d1m, b_h[NLAYER - 1].reshape(1, HID))

  q = agg(hp6, src1, dst3)
  wo = jnp.zeros((HID, HID), jnp.float32).at[:, :CLS].set(W_out)
  bo = jnp.zeros((1, HID), jnp.float32).at[0, :CLS].set(b_out)
  out128 = _out_call(q[0], q[1], hp6, d1m, wo, bo)
  return out128[:, :CLS]


# speedup vs baseline: 14.2606x; 1.0104x over previous
import functools

import jax
import jax.numpy as jnp
from jax import lax
from jax.experimental import pallas as pl
from jax.experimental.pallas import tpu as pltpu
from jax.experimental.pallas import tpu_sc as plsc

N = 10000
NPAD = 10240
E = 320000
HID = 128
CLS = 4
NLAYER = 5
CW = 16

NC = 2
NS = 16
NW = NC * NS
EP = E // NW
ECH = 80
NCH = EP // ECH
RPT = NPAD // NS
ZR = 128


def _make_agg():
  mesh = plsc.VectorSubcoreMesh(core_axis_name="c", subcore_axis_name="s")

  @functools.partial(
      pl.kernel,
      mesh=mesh,
      out_type=jax.ShapeDtypeStruct((NC, NPAD, HID), jnp.float32),
      scratch_types=[
          pltpu.VMEM((EP,), jnp.int32),
          pltpu.VMEM((NCH, ECH), jnp.int32),
          pltpu.VMEM((ECH, HID), jnp.float32),
          pltpu.VMEM((ECH, HID), jnp.float32),
          pltpu.VMEM_SHARED((NPAD, HID), jnp.float32),
          pltpu.SemaphoreType.DMA,
          pltpu.SemaphoreType.DMA,
          pltpu.SemaphoreType.DMA,
          pltpu.SemaphoreType.DMA,
      ],
  )
  def agg(rows_hbm, src_hbm, dst_hbm, out_hbm, src_v, dst_v, buf0, buf1,
          acc_sh, sem0, sem1, ssem0, ssem1):
    cid = lax.axis_index("c")
    sid = lax.axis_index("s")
    wid = sid * NC + cid

    def zero_row(i, carry):
      for j in range(HID // 16):
        buf0[i, pl.ds(j * 16, 16)] = jnp.zeros((16,), jnp.float32)
      return carry

    lax.fori_loop(0, ECH, zero_row, 0)
    for r in range(RPT // ECH):
      pltpu.sync_copy(buf0, acc_sh.at[pl.ds(sid * RPT + r * ECH, ECH)])
    pltpu.sync_copy(src_hbm.at[pl.ds(wid * EP, EP)], src_v)
    pltpu.sync_copy(dst_hbm.at[wid], dst_v)
    plsc.subcore_barrier()

    pltpu.async_copy(rows_hbm.at[src_v.at[pl.ds(0, ECH)]], buf0, sem0)

    pltpu.async_copy(rows_hbm.at[src_v.at[pl.ds(ECH, ECH)]], buf1, sem1)

    def pair(kk, carry):
      k0 = 2 * kk
      pltpu.make_async_copy(rows_hbm.at[src_v.at[pl.ds(k0 * ECH, ECH)]], buf0, sem0).wait()
      pltpu.async_copy(buf0, acc_sh.at[dst_v.at[k0]], ssem0, add=True)
      pltpu.make_async_copy(rows_hbm.at[src_v.at[pl.ds((k0 + 1) * ECH, ECH)]], buf1, sem1).wait()
      pltpu.async_copy(buf1, acc_sh.at[dst_v.at[k0 + 1]], ssem1, add=True)
      pltpu.make_async_copy(buf0, acc_sh.at[dst_v.at[k0]], ssem0).wait()

      @pl.when(k0 + 2 < NCH)
      def _():
        pltpu.async_copy(rows_hbm.at[src_v.at[pl.ds((k0 + 2) * ECH, ECH)]], buf0, sem0)

      pltpu.make_async_copy(buf1, acc_sh.at[dst_v.at[k0 + 1]], ssem1).wait()

      @pl.when(k0 + 3 < NCH)
      def _():
        pltpu.async_copy(rows_hbm.at[src_v.at[pl.ds((k0 + 3) * ECH, ECH)]], buf1, sem1)

      return carry

    lax.fori_loop(0, (NCH - 1) // 2, pair, 0)
    pltpu.make_async_copy(rows_hbm.at[src_v.at[pl.ds((NCH - 1) * ECH, ECH)]], buf0, sem0).wait()
    pltpu.sync_copy(buf0, acc_sh.at[dst_v.at[NCH - 1]], add=True)

    plsc.subcore_barrier()
    pltpu.sync_copy(acc_sh.at[pl.ds(sid * RPT, RPT)],
                    out_hbm.at[cid, pl.ds(sid * RPT, RPT)])

  return agg


def _make_cnt():
  mesh = plsc.VectorSubcoreMesh(core_axis_name="c", subcore_axis_name="s")

  @functools.partial(
      pl.kernel,
      mesh=mesh,
      out_type=jax.ShapeDtypeStruct((NC, NPAD, CW), jnp.float32),
      scratch_types=[
          pltpu.VMEM((NCH, ECH), jnp.int32),
          pltpu.VMEM((ECH, CW), jnp.float32),
          pltpu.VMEM((ZR, CW), jnp.float32),
          pltpu.VMEM_SHARED((NPAD, CW), jnp.float32),
      ],
  )
  def cnt(dst_hbm, out_hbm, dst_v, one_v, zero_v, acc_sh):
    cid = lax.axis_index("c")
    sid = lax.axis_index("s")
    wid = sid * NC + cid

    def fill_one(i, carry):
      one_v[i, pl.ds(0, 16)] = jnp.full((16,), 1.0, jnp.float32)
      return carry

    lax.fori_loop(0, ECH, fill_one, 0)

    def fill_zero(i, carry):
      zero_v[i, pl.ds(0, 16)] = jnp.zeros((16,), jnp.float32)
      return carry

    lax.fori_loop(0, ZR, fill_zero, 0)
    for r in range(RPT // ZR):
      pltpu.sync_copy(zero_v, acc_sh.at[pl.ds(sid * RPT + r * ZR, ZR)])
    pltpu.sync_copy(dst_hbm.at[wid], dst_v)
    plsc.subcore_barrier()

    def chunk(k, carry):
      pltpu.sync_copy(one_v, acc_sh.at[dst_v.at[k]], add=True)
      return carry

    lax.fori_loop(0, NCH, chunk, 0)

    plsc.subcore_barrier()
    pltpu.sync_copy(acc_sh.at[pl.ds(sid * RPT, RPT)],
                    out_hbm.at[cid, pl.ds(sid * RPT, RPT)])

  return cnt


@functools.lru_cache(maxsize=None)
def _get_agg():
  return _make_agg()


@functools.lru_cache(maxsize=None)
def _get_cnt():
  return _make_cnt()


_TC_R = 1000
_GRID = N // _TC_R


def _row_spec(w=HID):
  return pl.BlockSpec((_TC_R, w), lambda i: (i, 0))


def _fixed_spec(a, b):
  return pl.BlockSpec((a, b), lambda i: (0, 0))


def _dis_body(c0_ref, c1_ref, d2_ref, d1_ref):
  cnt16 = c0_ref[...] + c1_ref[...]
  cnt = jnp.broadcast_to(cnt16[:, :1], (_TC_R, HID))
  d2_ref[...] = lax.rsqrt(cnt + 2.0)
  d1_ref[...] = lax.rsqrt(cnt + 1.0)


_dis_call = pl.pallas_call(
    _dis_body,
    grid=(_GRID,),
    in_specs=[_row_spec(CW), _row_spec(CW)],
    out_specs=[_row_spec(), _row_spec()],
    out_shape=[jax.ShapeDtypeStruct((N, HID), jnp.float32)] * 2,
)


def _in_body(x_ref, w_ref, d2_ref, hp_ref):
  hp_ref[...] = d2_ref[...] * jnp.dot(
      x_ref[...], w_ref[...], preferred_element_type=jnp.float32)


_in_call = pl.pallas_call(
    _in_body,
    grid=(_GRID,),
    in_specs=[_row_spec(), _fixed_spec(HID, HID), _row_spec()],
    out_specs=_row_spec(),
    out_shape=jax.ShapeDtypeStruct((N, HID), jnp.float32),
)


def _mid_body(p0_ref, p1_ref, hp_ref, d2_ref, b_ref, w_ref, o_ref):
  g = d2_ref[...] * (p0_ref[...] + p1_ref[...] + 2.0 * hp_ref[...]) + b_ref[...]
  g = jnp.maximum(g, 0.0)
  o_ref[...] = d2_ref[...] * jnp.dot(
      g, w_ref[...], preferred_element_type=jnp.float32)


_mid_call = pl.pallas_call(
    _mid_body,
    grid=(_GRID,),
    in_specs=[_row_spec(), _row_spec(), _row_spec(), _row_spec(),
              _fixed_spec(1, HID), _fixed_spec(HID, HID)],
    out_specs=_row_spec(),
    out_shape=jax.ShapeDtypeStruct((N, HID), jnp.float32),
)


def _pre_body(p0_ref, p1_ref, hp_ref, d2_ref, d1_ref, b_ref, o_ref):
  g = d2_ref[...] * (p0_ref[...] + p1_ref[...] + 2.0 * hp_ref[...]) + b_ref[...]
  o_ref[...] = d1_ref[...] * jnp.maximum(g, 0.0)


_pre_call = pl.pallas_call(
    _pre_body,
    grid=(_GRID,),
    in_specs=[_row_spec(), _row_spec(), _row_spec(), _row_spec(), _row_spec(),
              _fixed_spec(1, HID)],
    out_specs=_row_spec(),
    out_shape=jax.ShapeDtypeStruct((N, HID), jnp.float32),
)


def _out_body(q0_ref, q1_ref, hp_ref, d1_ref, w_ref, b_ref, o_ref):
  agg = d1_ref[...] * (q0_ref[...] + q1_ref[...] + hp_ref[...])
  o_ref[...] = jnp.dot(
      agg, w_ref[...], preferred_element_type=jnp.float32) + b_ref[...]


_out_call = pl.pallas_call(
    _out_body,
    grid=(_GRID,),
    in_specs=[_row_spec(), _row_spec(), _row_spec(), _row_spec(),
              _fixed_spec(HID, HID), _fixed_spec(1, HID)],
    out_specs=_row_spec(),
    out_shape=jax.ShapeDtypeStruct((N, HID), jnp.float32),
)


def kernel(x, edge_index, W_in, b_in, W_h, b_h, W_out, b_out):
  src1 = edge_index[0]
  dst3 = edge_index[1].reshape(NW, NCH, ECH)

  agg = _get_agg()
  cnt = _get_cnt()(dst3)
  d2m, d1m = _dis_call(cnt[0, :, :], cnt[1, :, :])

  hp = _in_call(x, W_in, d2m)
  biases = [b_in] + [b_h[i] for i in range(NLAYER - 1)]
  for j in range(NLAYER):
    p = agg(hp, src1, dst3)
    hp = _mid_call(p[0], p[1], hp, d2m, biases[j].reshape(1, HID), W_h[j])

  p = agg(hp, src1, dst3)
  hp6 = _pre_call(p[0], p[1], hp, d2m, d1m, b_h[NLAYER - 1].reshape(1, HID))

  q = agg(hp6, src1, dst3)
  wo = jnp.zeros((HID, HID), jnp.float32).at[:, :CLS].set(W_out)
  bo = jnp.zeros((1, HID), jnp.float32).at[0, :CLS].set(b_out)
  out128 = _out_call(q[0], q[1], hp6, d1m, wo, bo)
  return out128[:, :CLS]


# speedup vs baseline: 14.3141x; 1.0038x over previous
import functools

import jax
import jax.numpy as jnp
from jax import lax
from jax.experimental import pallas as pl
from jax.experimental.pallas import tpu as pltpu
from jax.experimental.pallas import tpu_sc as plsc

N = 10000
NPAD = 10240
E = 320000
HID = 128
CLS = 4
NLAYER = 5
CW = 16

NC = 2
NS = 16
NW = NC * NS
EP = E // NW
ECH = 80
NCH = EP // ECH
RPT = NPAD // NS
ZR = 128


def _make_agg(h):
  mesh = plsc.VectorSubcoreMesh(core_axis_name="c", subcore_axis_name="s")
  params = None

  @functools.partial(
      pl.kernel,
      mesh=mesh,
      compiler_params=params,
      out_type=jax.ShapeDtypeStruct((NC, NPAD, h), jnp.float32),
      scratch_types=[
          pltpu.VMEM((EP,), jnp.int32),
          pltpu.VMEM((NCH, ECH), jnp.int32),
          pltpu.VMEM((ECH, h), jnp.float32),
          pltpu.VMEM((ECH, h), jnp.float32),
          pltpu.VMEM_SHARED((NPAD, h), jnp.float32),
          pltpu.SemaphoreType.DMA,
          pltpu.SemaphoreType.DMA,
          pltpu.SemaphoreType.DMA,
          pltpu.SemaphoreType.DMA,
      ],
  )
  def agg(rows_hbm, src_hbm, dst_hbm, out_hbm, src_v, dst_v, buf0, buf1,
          acc_sh, sem0, sem1, ssem0, ssem1):
    cid = lax.axis_index("c")
    sid = lax.axis_index("s")
    wid = sid * NC + cid

    def zero_row(i, carry):
      for j in range(h // 16):
        buf0[i, pl.ds(j * 16, 16)] = jnp.zeros((16,), jnp.float32)
      return carry

    lax.fori_loop(0, ECH, zero_row, 0)
    for r in range(RPT // ECH):
      pltpu.sync_copy(buf0, acc_sh.at[pl.ds(sid * RPT + r * ECH, ECH)])
    pltpu.sync_copy(src_hbm.at[pl.ds(wid * EP, EP)], src_v)
    pltpu.sync_copy(dst_hbm.at[wid], dst_v)
    plsc.subcore_barrier()

    pltpu.async_copy(rows_hbm.at[src_v.at[pl.ds(0, ECH)]], buf0, sem0)

    pltpu.async_copy(rows_hbm.at[src_v.at[pl.ds(ECH, ECH)]], buf1, sem1)

    def pair(kk, carry):
      k0 = 2 * kk
      pltpu.make_async_copy(rows_hbm.at[src_v.at[pl.ds(k0 * ECH, ECH)]], buf0, sem0).wait()
      pltpu.async_copy(buf0, acc_sh.at[dst_v.at[k0]], ssem0, add=True)
      pltpu.make_async_copy(rows_hbm.at[src_v.at[pl.ds((k0 + 1) * ECH, ECH)]], buf1, sem1).wait()
      pltpu.async_copy(buf1, acc_sh.at[dst_v.at[k0 + 1]], ssem1, add=True)
      pltpu.make_async_copy(buf0, acc_sh.at[dst_v.at[k0]], ssem0).wait()

      @pl.when(k0 + 2 < NCH)
      def _():
        pltpu.async_copy(rows_hbm.at[src_v.at[pl.ds((k0 + 2) * ECH, ECH)]], buf0, sem0)

      pltpu.make_async_copy(buf1, acc_sh.at[dst_v.at[k0 + 1]], ssem1).wait()

      @pl.when(k0 + 3 < NCH)
      def _():
        pltpu.async_copy(rows_hbm.at[src_v.at[pl.ds((k0 + 3) * ECH, ECH)]], buf1, sem1)

      return carry

    lax.fori_loop(0, (NCH - 1) // 2, pair, 0)
    pltpu.make_async_copy(rows_hbm.at[src_v.at[pl.ds((NCH - 1) * ECH, ECH)]], buf0, sem0).wait()
    pltpu.sync_copy(buf0, acc_sh.at[dst_v.at[NCH - 1]], add=True)

    plsc.subcore_barrier()
    pltpu.sync_copy(acc_sh.at[pl.ds(sid * RPT, RPT)],
                    out_hbm.at[cid, pl.ds(sid * RPT, RPT)])

  return agg


def _make_cnt():
  mesh = plsc.VectorSubcoreMesh(core_axis_name="c", subcore_axis_name="s")

  @functools.partial(
      pl.kernel,
      mesh=mesh,
      out_type=jax.ShapeDtypeStruct((NC, NPAD, CW), jnp.float32),
      scratch_types=[
          pltpu.VMEM((NCH, ECH), jnp.int32),
          pltpu.VMEM((ECH, CW), jnp.float32),
          pltpu.VMEM((ZR, CW), jnp.float32),
          pltpu.VMEM_SHARED((NPAD, CW), jnp.float32),
      ],
  )
  def cnt(dst_hbm, out_hbm, dst_v, one_v, zero_v, acc_sh):
    cid = lax.axis_index("c")
    sid = lax.axis_index("s")
    wid = sid * NC + cid

    def fill_one(i, carry):
      one_v[i, pl.ds(0, 16)] = jnp.full((16,), 1.0, jnp.float32)
      return carry

    lax.fori_loop(0, ECH, fill_one, 0)

    def fill_zero(i, carry):
      zero_v[i, pl.ds(0, 16)] = jnp.zeros((16,), jnp.float32)
      return carry

    lax.fori_loop(0, ZR, fill_zero, 0)
    for r in range(RPT // ZR):
      pltpu.sync_copy(zero_v, acc_sh.at[pl.ds(sid * RPT + r * ZR, ZR)])
    pltpu.sync_copy(dst_hbm.at[wid], dst_v)
    plsc.subcore_barrier()

    def chunk(k, carry):
      pltpu.sync_copy(one_v, acc_sh.at[dst_v.at[k]], add=True)
      return carry

    lax.fori_loop(0, NCH, chunk, 0)

    plsc.subcore_barrier()
    pltpu.sync_copy(acc_sh.at[pl.ds(sid * RPT, RPT)],
                    out_hbm.at[cid, pl.ds(sid * RPT, RPT)])

  return cnt


@functools.lru_cache(maxsize=None)
def _get_agg(h):
  return _make_agg(h)


@functools.lru_cache(maxsize=None)
def _get_cnt():
  return _make_cnt()


_TC_R = 1000
_GRID = N // _TC_R


def _row_spec(w=HID):
  return pl.BlockSpec((_TC_R, w), lambda i: (i, 0))


def _fixed_spec(a, b):
  return pl.BlockSpec((a, b), lambda i: (0, 0))


def _dis_body(c0_ref, c1_ref, mm_ref, d2_ref, d1_ref, hp_ref):
  cnt16 = c0_ref[...] + c1_ref[...]
  cnt = jnp.broadcast_to(cnt16[:, :1], (_TC_R, HID))
  d2 = lax.rsqrt(cnt + 2.0)
  d2_ref[...] = d2
  d1_ref[...] = lax.rsqrt(cnt + 1.0)
  hp_ref[...] = d2 * mm_ref[...]


_dis_call = pl.pallas_call(
    _dis_body,
    grid=(_GRID,),
    in_specs=[_row_spec(CW), _row_spec(CW), _row_spec()],
    out_specs=[_row_spec(), _row_spec(), _row_spec()],
    out_shape=[jax.ShapeDtypeStruct((N, HID), jnp.float32)] * 3,
)


def _mm_body(x_ref, w_ref, o_ref):
  o_ref[...] = jnp.dot(
      x_ref[...], w_ref[...], preferred_element_type=jnp.float32)


_mm_call = pl.pallas_call(
    _mm_body,
    grid=(_GRID,),
    in_specs=[_row_spec(), _fixed_spec(HID, HID)],
    out_specs=_row_spec(),
    out_shape=jax.ShapeDtypeStruct((N, HID), jnp.float32),
)


def _out_body(q0_ref, q1_ref, hp_ref, d1_ref, w_ref, b_ref, o_ref):
  agg = d1_ref[...] * (q0_ref[...] + q1_ref[...] + hp_ref[...])
  o_ref[...] = jnp.dot(
      agg, w_ref[...], preferred_element_type=jnp.float32) + b_ref[...]


_out_call = pl.pallas_call(
    _out_body,
    grid=(_GRID,),
    in_specs=[_row_spec(), _row_spec(), _row_spec(), _row_spec(),
              _fixed_spec(HID, HID), _fixed_spec(1, HID)],
    out_specs=_row_spec(),
    out_shape=jax.ShapeDtypeStruct((N, HID), jnp.float32),
)


def _mid_body(p0_ref, p1_ref, hp_ref, d2_ref, b_ref, w_ref, o_ref):
  g = d2_ref[...] * (p0_ref[...] + p1_ref[...] + 2.0 * hp_ref[...]) + b_ref[...]
  g = jnp.maximum(g, 0.0)
  o_ref[...] = d2_ref[...] * jnp.dot(
      g, w_ref[...], preferred_element_type=jnp.float32)


_mid_call = pl.pallas_call(
    _mid_body,
    grid=(_GRID,),
    in_specs=[_row_spec(), _row_spec(), _row_spec(), _row_spec(),
              _fixed_spec(1, HID), _fixed_spec(HID, HID)],
    out_specs=_row_spec(),
    out_shape=jax.ShapeDtypeStruct((N, HID), jnp.float32),
)


def _pre_body(p0_ref, p1_ref, hp_ref, d2_ref, d1_ref, b_ref, o_ref):
  g = d2_ref[...] * (p0_ref[...] + p1_ref[...] + 2.0 * hp_ref[...]) + b_ref[...]
  o_ref[...] = d1_ref[...] * jnp.maximum(g, 0.0)


_pre_call = pl.pallas_call(
    _pre_body,
    grid=(_GRID,),
    in_specs=[_row_spec(), _row_spec(), _row_spec(), _row_spec(), _row_spec(),
              _fixed_spec(1, HID)],
    out_specs=_row_spec(),
    out_shape=jax.ShapeDtypeStruct((N, HID), jnp.float32),
)


def kernel(x, edge_index, W_in, b_in, W_h, b_h, W_out, b_out):
  src1 = edge_index[0]
  dst3 = edge_index[1].reshape(NW, NCH, ECH)

  agg = _get_agg(HID)

  mm = _mm_call(x, W_in)
  cnt = _get_cnt()(dst3)
  d2m, d1m, hp = _dis_call(cnt[0, :, :], cnt[1, :, :], mm)

  biases = [b_in] + [b_h[i] for i in range(NLAYER - 1)]
  for j in range(NLAYER):
    p = agg(hp, src1, dst3)
    hp = _mid_call(p[0], p[1], hp, d2m, biases[j].reshape(1, HID), W_h[j])

  p = agg(hp, src1, dst3)
  hp6 = _pre_call(p[0], p[1], hp, d2m, d1m, b_h[NLAYER - 1].reshape(1, HID))

  q = agg(hp6, src1, dst3)
  wo = jnp.zeros((HID, HID), jnp.float32).at[:, :CLS].set(W_out)
  bo = jnp.zeros((1, HID), jnp.float32).at[0, :CLS].set(b_out)
  out128 = _out_call(q[0], q[1], hp6, d1m, wo, bo)
  return out128[:, :CLS]


# speedup vs baseline: 14.3860x; 1.0050x over previous
import functools

import jax
import jax.numpy as jnp
from jax import lax
from jax.experimental import pallas as pl
from jax.experimental.pallas import tpu as pltpu
from jax.experimental.pallas import tpu_sc as plsc

N = 10000
NPAD = 10240
E = 320000
HID = 128
CLS = 4
NLAYER = 5
CW = 16

NC = 2
NS = 16
NW = NC * NS
EP = E // NW
ECH = 80
NCH = EP // ECH
RPT = NPAD // NS
ZR = 128


def _make_agg(h):
  mesh = plsc.VectorSubcoreMesh(core_axis_name="c", subcore_axis_name="s")
  params = None

  @functools.partial(
      pl.kernel,
      mesh=mesh,
      compiler_params=params,
      out_type=jax.ShapeDtypeStruct((NC, NPAD, h), jnp.float32),
      scratch_types=[
          pltpu.VMEM((EP,), jnp.int32),
          pltpu.VMEM((NCH, ECH), jnp.int32),
          pltpu.VMEM((ECH, h), jnp.float32),
          pltpu.VMEM((ECH, h), jnp.float32),
          pltpu.VMEM_SHARED((NPAD, h), jnp.float32),
          pltpu.SemaphoreType.DMA,
          pltpu.SemaphoreType.DMA,
          pltpu.SemaphoreType.DMA,
          pltpu.SemaphoreType.DMA,
      ],
  )
  def agg(rows_hbm, src_hbm, dst_hbm, out_hbm, src_v, dst_v, buf0, buf1,
          acc_sh, sem0, sem1, ssem0, ssem1):
    cid = lax.axis_index("c")
    sid = lax.axis_index("s")
    wid = sid * NC + cid

    def zero_row(i, carry):
      for j in range(h // 16):
        buf0[i, pl.ds(j * 16, 16)] = jnp.zeros((16,), jnp.float32)
      return carry

    lax.fori_loop(0, ECH, zero_row, 0)
    for r in range(RPT // ECH):
      pltpu.sync_copy(buf0, acc_sh.at[pl.ds(sid * RPT + r * ECH, ECH)])
    pltpu.sync_copy(src_hbm.at[pl.ds(wid * EP, EP)], src_v)
    pltpu.sync_copy(dst_hbm.at[wid], dst_v)
    plsc.subcore_barrier()

    pltpu.async_copy(rows_hbm.at[src_v.at[pl.ds(0, ECH)]], buf0, sem0)

    pltpu.async_copy(rows_hbm.at[src_v.at[pl.ds(ECH, ECH)]], buf1, sem1)

    def pair(kk, carry):
      k0 = 2 * kk
      pltpu.make_async_copy(rows_hbm.at[src_v.at[pl.ds(k0 * ECH, ECH)]], buf0, sem0).wait()
      pltpu.async_copy(buf0, acc_sh.at[dst_v.at[k0]], ssem0, add=True)
      pltpu.make_async_copy(rows_hbm.at[src_v.at[pl.ds((k0 + 1) * ECH, ECH)]], buf1, sem1).wait()
      pltpu.async_copy(buf1, acc_sh.at[dst_v.at[k0 + 1]], ssem1, add=True)
      pltpu.make_async_copy(buf0, acc_sh.at[dst_v.at[k0]], ssem0).wait()

      @pl.when(k0 + 2 < NCH)
      def _():
        pltpu.async_copy(rows_hbm.at[src_v.at[pl.ds((k0 + 2) * ECH, ECH)]], buf0, sem0)

      pltpu.make_async_copy(buf1, acc_sh.at[dst_v.at[k0 + 1]], ssem1).wait()

      @pl.when(k0 + 3 < NCH)
      def _():
        pltpu.async_copy(rows_hbm.at[src_v.at[pl.ds((k0 + 3) * ECH, ECH)]], buf1, sem1)

      return carry

    lax.fori_loop(0, (NCH - 1) // 2, pair, 0)
    pltpu.make_async_copy(rows_hbm.at[src_v.at[pl.ds((NCH - 1) * ECH, ECH)]], buf0, sem0).wait()
    pltpu.sync_copy(buf0, acc_sh.at[dst_v.at[NCH - 1]], add=True)

    plsc.subcore_barrier()
    pltpu.sync_copy(acc_sh.at[pl.ds(sid * RPT, RPT)],
                    out_hbm.at[cid, pl.ds(sid * RPT, RPT)])

  return agg


def _make_cnt():
  mesh = plsc.VectorSubcoreMesh(core_axis_name="c", subcore_axis_name="s")

  @functools.partial(
      pl.kernel,
      mesh=mesh,
      out_type=jax.ShapeDtypeStruct((NC, NPAD, CW), jnp.float32),
      scratch_types=[
          pltpu.VMEM((NCH, ECH), jnp.int32),
          pltpu.VMEM((ECH, CW), jnp.float32),
          pltpu.VMEM((ZR, CW), jnp.float32),
          pltpu.VMEM_SHARED((NPAD, CW), jnp.float32),
          pltpu.SemaphoreType.DMA,
      ],
  )
  def cnt(dst_hbm, out_hbm, dst_v, one_v, zero_v, acc_sh, csem):
    cid = lax.axis_index("c")
    sid = lax.axis_index("s")
    wid = sid * NC + cid

    def fill_one(i, carry):
      one_v[i, pl.ds(0, 16)] = jnp.full((16,), 1.0, jnp.float32)
      return carry

    lax.fori_loop(0, ECH, fill_one, 0)

    def fill_zero(i, carry):
      zero_v[i, pl.ds(0, 16)] = jnp.zeros((16,), jnp.float32)
      return carry

    lax.fori_loop(0, ZR, fill_zero, 0)
    for r in range(RPT // ZR):
      pltpu.sync_copy(zero_v, acc_sh.at[pl.ds(sid * RPT + r * ZR, ZR)])
    pltpu.sync_copy(dst_hbm.at[wid], dst_v)
    plsc.subcore_barrier()

    def chunk4(kk, carry):
      k0 = 4 * kk
      for j in range(4):
        pltpu.async_copy(one_v, acc_sh.at[dst_v.at[k0 + j]], csem, add=True)
      for j in range(4):
        pltpu.make_async_copy(one_v, acc_sh.at[dst_v.at[k0 + j]], csem).wait()
      return carry

    lax.fori_loop(0, NCH // 4, chunk4, 0)
    pltpu.sync_copy(one_v, acc_sh.at[dst_v.at[NCH - 1]], add=True)

    plsc.subcore_barrier()
    pltpu.sync_copy(acc_sh.at[pl.ds(sid * RPT, RPT)],
                    out_hbm.at[cid, pl.ds(sid * RPT, RPT)])

  return cnt


@functools.lru_cache(maxsize=None)
def _get_agg(h):
  return _make_agg(h)


@functools.lru_cache(maxsize=None)
def _get_cnt():
  return _make_cnt()


_TC_R = 1000
_GRID = N // _TC_R


def _row_spec(w=HID):
  return pl.BlockSpec((_TC_R, w), lambda i: (i, 0))


def _fixed_spec(a, b):
  return pl.BlockSpec((a, b), lambda i: (0, 0))


def _dis_body(c0_ref, c1_ref, mm_ref, d2_ref, d1_ref, hp_ref):
  cnt16 = c0_ref[...] + c1_ref[...]
  cnt = jnp.broadcast_to(cnt16[:, :1], (_TC_R, HID))
  d2 = lax.rsqrt(cnt + 2.0)
  d2_ref[...] = d2
  d1_ref[...] = lax.rsqrt(cnt + 1.0)
  hp_ref[...] = d2 * mm_ref[...]


_dis_call = pl.pallas_call(
    _dis_body,
    grid=(_GRID,),
    in_specs=[_row_spec(CW), _row_spec(CW), _row_spec()],
    out_specs=[_row_spec(), _row_spec(), _row_spec()],
    out_shape=[jax.ShapeDtypeStruct((N, HID), jnp.float32)] * 3,
)


def _mm_body(x_ref, w_ref, o_ref):
  o_ref[...] = jnp.dot(
      x_ref[...], w_ref[...], preferred_element_type=jnp.float32)


_mm_call = pl.pallas_call(
    _mm_body,
    grid=(_GRID,),
    in_specs=[_row_spec(), _fixed_spec(HID, HID)],
    out_specs=_row_spec(),
    out_shape=jax.ShapeDtypeStruct((N, HID), jnp.float32),
)


def _out_body(q0_ref, q1_ref, hp_ref, d1_ref, w_ref, b_ref, o_ref):
  agg = d1_ref[...] * (q0_ref[...] + q1_ref[...] + hp_ref[...])
  o_ref[...] = jnp.dot(
      agg, w_ref[...], preferred_element_type=jnp.float32) + b_ref[...]


_out_call = pl.pallas_call(
    _out_body,
    grid=(_GRID,),
    in_specs=[_row_spec(), _row_spec(), _row_spec(), _row_spec(),
              _fixed_spec(HID, HID), _fixed_spec(1, HID)],
    out_specs=_row_spec(),
    out_shape=jax.ShapeDtypeStruct((N, HID), jnp.float32),
)


def _mid_body(p0_ref, p1_ref, hp_ref, d2_ref, b_ref, w_ref, o_ref):
  g = d2_ref[...] * (p0_ref[...] + p1_ref[...] + 2.0 * hp_ref[...]) + b_ref[...]
  g = jnp.maximum(g, 0.0)
  o_ref[...] = d2_ref[...] * jnp.dot(
      g, w_ref[...], preferred_element_type=jnp.float32)


_mid_call = pl.pallas_call(
    _mid_body,
    grid=(_GRID,),
    in_specs=[_row_spec(), _row_spec(), _row_spec(), _row_spec(),
              _fixed_spec(1, HID), _fixed_spec(HID, HID)],
    out_specs=_row_spec(),
    out_shape=jax.ShapeDtypeStruct((N, HID), jnp.float32),
)


def _pre_body(p0_ref, p1_ref, hp_ref, d2_ref, d1_ref, b_ref, o_ref):
  g = d2_ref[...] * (p0_ref[...] + p1_ref[...] + 2.0 * hp_ref[...]) + b_ref[...]
  o_ref[...] = d1_ref[...] * jnp.maximum(g, 0.0)


_pre_call = pl.pallas_call(
    _pre_body,
    grid=(_GRID,),
    in_specs=[_row_spec(), _row_spec(), _row_spec(), _row_spec(), _row_spec(),
              _fixed_spec(1, HID)],
    out_specs=_row_spec(),
    out_shape=jax.ShapeDtypeStruct((N, HID), jnp.float32),
)


def kernel(x, edge_index, W_in, b_in, W_h, b_h, W_out, b_out):
  src1 = edge_index[0]
  dst3 = edge_index[1].reshape(NW, NCH, ECH)

  agg = _get_agg(HID)

  mm = _mm_call(x, W_in)
  cnt = _get_cnt()(dst3)
  d2m, d1m, hp = _dis_call(cnt[0, :, :], cnt[1, :, :], mm)

  biases = [b_in] + [b_h[i] for i in range(NLAYER - 1)]
  for j in range(NLAYER):
    p = agg(hp, src1, dst3)
    hp = _mid_call(p[0], p[1], hp, d2m, biases[j].reshape(1, HID), W_h[j])

  p = agg(hp, src1, dst3)
  hp6 = _pre_call(p[0], p[1], hp, d2m, d1m, b_h[NLAYER - 1].reshape(1, HID))

  q = agg(hp6, src1, dst3)
  wo = jnp.zeros((HID, HID), jnp.float32).at[:, :CLS].set(W_out)
  bo = jnp.zeros((1, HID), jnp.float32).at[0, :CLS].set(b_out)
  out128 = _out_call(q[0], q[1], hp6, d1m, wo, bo)
  return out128[:, :CLS]


# speedup vs baseline: 14.6398x; 1.0176x over previous
import functools

import jax
import jax.numpy as jnp
from jax import lax
from jax.experimental import pallas as pl
from jax.experimental.pallas import tpu as pltpu
from jax.experimental.pallas import tpu_sc as plsc

N = 10000
NPAD = 10240
E = 320000
HID = 128
CLS = 4
NLAYER = 5
CW = 16

NC = 2
NS = 16
NW = NC * NS
EP = E // NW
ECH = 96
NCH = 104
EMAIN = NCH * ECH
ETAIL = EP - EMAIN
RPT = NPAD // NS
ZR = 128


def _make_agg(h):
  mesh = plsc.VectorSubcoreMesh(core_axis_name="c", subcore_axis_name="s")
  params = None

  @functools.partial(
      pl.kernel,
      mesh=mesh,
      compiler_params=params,
      out_type=jax.ShapeDtypeStruct((NC, NPAD, h), jnp.float32),
      scratch_types=[
          pltpu.VMEM((EMAIN,), jnp.int32),
          pltpu.VMEM((NCH, ECH), jnp.int32),
          pltpu.VMEM((ETAIL,), jnp.int32),
          pltpu.VMEM((ETAIL,), jnp.int32),
          pltpu.VMEM((ECH, h), jnp.float32),
          pltpu.VMEM((ECH, h), jnp.float32),
          pltpu.VMEM_SHARED((NPAD, h), jnp.float32),
          pltpu.SemaphoreType.DMA,
          pltpu.SemaphoreType.DMA,
          pltpu.SemaphoreType.DMA,
          pltpu.SemaphoreType.DMA,
      ],
  )
  def agg(rows_hbm, src_hbm, dst_hbm, tsrc_hbm, tdst_hbm, out_hbm, src_v,
          dst_v, tsrc_v, tdst_v, buf0, buf1, acc_sh, sem0, sem1, ssem0, ssem1):
    cid = lax.axis_index("c")
    sid = lax.axis_index("s")
    wid = sid * NC + cid

    def zero_row(i, carry):
      for j in range(h // 16):
        buf0[i, pl.ds(j * 16, 16)] = jnp.zeros((16,), jnp.float32)
      return carry

    lax.fori_loop(0, ECH, zero_row, 0)
    for r in range(RPT // ECH):
      pltpu.sync_copy(buf0, acc_sh.at[pl.ds(sid * RPT + r * ECH, ECH)])
    if RPT % ECH:
      pltpu.sync_copy(
          buf0.at[pl.ds(0, RPT % ECH)],
          acc_sh.at[pl.ds(sid * RPT + (RPT // ECH) * ECH, RPT % ECH)])
    pltpu.sync_copy(src_hbm.at[pl.ds(wid * EP, EMAIN)], src_v)
    pltpu.sync_copy(dst_hbm.at[wid], dst_v)
    pltpu.sync_copy(tsrc_hbm.at[pl.ds(wid * ETAIL, ETAIL)], tsrc_v)
    pltpu.sync_copy(tdst_hbm.at[pl.ds(wid * ETAIL, ETAIL)], tdst_v)
    plsc.subcore_barrier()

    pltpu.async_copy(rows_hbm.at[src_v.at[pl.ds(0, ECH)]], buf0, sem0)

    pltpu.async_copy(rows_hbm.at[src_v.at[pl.ds(ECH, ECH)]], buf1, sem1)

    def pair(kk, carry):
      k0 = 2 * kk
      pltpu.make_async_copy(rows_hbm.at[src_v.at[pl.ds(k0 * ECH, ECH)]], buf0, sem0).wait()
      pltpu.async_copy(buf0, acc_sh.at[dst_v.at[k0]], ssem0, add=True)
      pltpu.make_async_copy(rows_hbm.at[src_v.at[pl.ds((k0 + 1) * ECH, ECH)]], buf1, sem1).wait()
      pltpu.async_copy(buf1, acc_sh.at[dst_v.at[k0 + 1]], ssem1, add=True)
      pltpu.make_async_copy(buf0, acc_sh.at[dst_v.at[k0]], ssem0).wait()

      @pl.when(k0 + 2 < NCH)
      def _():
        pltpu.async_copy(rows_hbm.at[src_v.at[pl.ds((k0 + 2) * ECH, ECH)]], buf0, sem0)

      pltpu.make_async_copy(buf1, acc_sh.at[dst_v.at[k0 + 1]], ssem1).wait()

      @pl.when(k0 + 3 < NCH)
      def _():
        pltpu.async_copy(rows_hbm.at[src_v.at[pl.ds((k0 + 3) * ECH, ECH)]], buf1, sem1)

      return carry

    lax.fori_loop(0, NCH // 2, pair, 0)
    pltpu.async_copy(rows_hbm.at[tsrc_v], buf0.at[pl.ds(0, ETAIL)], sem0)
    pltpu.make_async_copy(rows_hbm.at[tsrc_v], buf0.at[pl.ds(0, ETAIL)], sem0).wait()
    pltpu.sync_copy(buf0.at[pl.ds(0, ETAIL)], acc_sh.at[tdst_v], add=True)

    plsc.subcore_barrier()
    pltpu.sync_copy(acc_sh.at[pl.ds(sid * RPT, RPT)],
                    out_hbm.at[cid, pl.ds(sid * RPT, RPT)])

  return agg


def _make_cnt():
  mesh = plsc.VectorSubcoreMesh(core_axis_name="c", subcore_axis_name="s")

  @functools.partial(
      pl.kernel,
      mesh=mesh,
      out_type=jax.ShapeDtypeStruct((NC, NPAD, CW), jnp.float32),
      scratch_types=[
          pltpu.VMEM((NCH, ECH), jnp.int32),
          pltpu.VMEM((ETAIL,), jnp.int32),
          pltpu.VMEM((ECH, CW), jnp.float32),
          pltpu.VMEM((ZR, CW), jnp.float32),
          pltpu.VMEM_SHARED((NPAD, CW), jnp.float32),
          pltpu.SemaphoreType.DMA,
      ],
  )
  def cnt(dst_hbm, tdst_hbm, out_hbm, dst_v, tdst_v, one_v, zero_v, acc_sh,
          csem):
    cid = lax.axis_index("c")
    sid = lax.axis_index("s")
    wid = sid * NC + cid

    def fill_one(i, carry):
      one_v[i, pl.ds(0, 16)] = jnp.full((16,), 1.0, jnp.float32)
      return carry

    lax.fori_loop(0, ECH, fill_one, 0)

    def fill_zero(i, carry):
      zero_v[i, pl.ds(0, 16)] = jnp.zeros((16,), jnp.float32)
      return carry

    lax.fori_loop(0, ZR, fill_zero, 0)
    for r in range(RPT // ZR):
      pltpu.sync_copy(zero_v, acc_sh.at[pl.ds(sid * RPT + r * ZR, ZR)])
    pltpu.sync_copy(dst_hbm.at[wid], dst_v)
    pltpu.sync_copy(tdst_hbm.at[pl.ds(wid * ETAIL, ETAIL)], tdst_v)
    plsc.subcore_barrier()

    def chunk4(kk, carry):
      k0 = 4 * kk
      for j in range(4):
        pltpu.async_copy(one_v, acc_sh.at[dst_v.at[k0 + j]], csem, add=True)
      for j in range(4):
        pltpu.make_async_copy(one_v, acc_sh.at[dst_v.at[k0 + j]], csem).wait()
      return carry

    lax.fori_loop(0, NCH // 4, chunk4, 0)
    pltpu.sync_copy(one_v.at[pl.ds(0, ETAIL)], acc_sh.at[tdst_v], add=True)

    plsc.subcore_barrier()
    pltpu.sync_copy(acc_sh.at[pl.ds(sid * RPT, RPT)],
                    out_hbm.at[cid, pl.ds(sid * RPT, RPT)])

  return cnt


@functools.lru_cache(maxsize=None)
def _get_agg(h):
  return _make_agg(h)


@functools.lru_cache(maxsize=None)
def _get_cnt():
  return _make_cnt()


_TC_R = 1000
_GRID = N // _TC_R


def _row_spec(w=HID):
  return pl.BlockSpec((_TC_R, w), lambda i: (i, 0))


def _fixed_spec(a, b):
  return pl.BlockSpec((a, b), lambda i: (0, 0))


def _dis_body(c0_ref, c1_ref, mm_ref, d2_ref, d1_ref, hp_ref):
  cnt16 = c0_ref[...] + c1_ref[...]
  cnt = jnp.broadcast_to(cnt16[:, :1], (_TC_R, HID))
  d2 = lax.rsqrt(cnt + 2.0)
  d2_ref[...] = d2
  d1_ref[...] = lax.rsqrt(cnt + 1.0)
  hp_ref[...] = d2 * mm_ref[...]


_dis_call = pl.pallas_call(
    _dis_body,
    grid=(_GRID,),
    in_specs=[_row_spec(CW), _row_spec(CW), _row_spec()],
    out_specs=[_row_spec(), _row_spec(), _row_spec()],
    out_shape=[jax.ShapeDtypeStruct((N, HID), jnp.float32)] * 3,
)


def _mm_body(x_ref, w_ref, o_ref):
  o_ref[...] = jnp.dot(
      x_ref[...], w_ref[...], preferred_element_type=jnp.float32)


_mm_call = pl.pallas_call(
    _mm_body,
    grid=(_GRID,),
    in_specs=[_row_spec(), _fixed_spec(HID, HID)],
    out_specs=_row_spec(),
    out_shape=jax.ShapeDtypeStruct((N, HID), jnp.float32),
)


def _out_body(q0_ref, q1_ref, hp_ref, d1_ref, w_ref, b_ref, o_ref):
  agg = d1_ref[...] * (q0_ref[...] + q1_ref[...] + hp_ref[...])
  o_ref[...] = jnp.dot(
      agg, w_ref[...], preferred_element_type=jnp.float32) + b_ref[...]


_out_call = pl.pallas_call(
    _out_body,
    grid=(_GRID,),
    in_specs=[_row_spec(), _row_spec(), _row_spec(), _row_spec(),
              _fixed_spec(HID, HID), _fixed_spec(1, HID)],
    out_specs=_row_spec(),
    out_shape=jax.ShapeDtypeStruct((N, HID), jnp.float32),
)


def _mid_body(p0_ref, p1_ref, hp_ref, d2_ref, b_ref, w_ref, o_ref):
  g = d2_ref[...] * (p0_ref[...] + p1_ref[...] + 2.0 * hp_ref[...]) + b_ref[...]
  g = jnp.maximum(g, 0.0)
  o_ref[...] = d2_ref[...] * jnp.dot(
      g, w_ref[...], preferred_element_type=jnp.float32)


_mid_call = pl.pallas_call(
    _mid_body,
    grid=(_GRID,),
    in_specs=[_row_spec(), _row_spec(), _row_spec(), _row_spec(),
              _fixed_spec(1, HID), _fixed_spec(HID, HID)],
    out_specs=_row_spec(),
    out_shape=jax.ShapeDtypeStruct((N, HID), jnp.float32),
)


def _pre_body(p0_ref, p1_ref, hp_ref, d2_ref, d1_ref, b_ref, o_ref):
  g = d2_ref[...] * (p0_ref[...] + p1_ref[...] + 2.0 * hp_ref[...]) + b_ref[...]
  o_ref[...] = d1_ref[...] * jnp.maximum(g, 0.0)


_pre_call = pl.pallas_call(
    _pre_body,
    grid=(_GRID,),
    in_specs=[_row_spec(), _row_spec(), _row_spec(), _row_spec(), _row_spec(),
              _fixed_spec(1, HID)],
    out_specs=_row_spec(),
    out_shape=jax.ShapeDtypeStruct((N, HID), jnp.float32),
)


def kernel(x, edge_index, W_in, b_in, W_h, b_h, W_out, b_out):
  src1 = edge_index[0]
  dstw = edge_index[1].reshape(NW, EP)
  dst3 = dstw[:, :EMAIN].reshape(NW, NCH, ECH)
  tsrc = edge_index[0].reshape(NW, EP)[:, EMAIN:].reshape(NW * ETAIL)
  tdst = dstw[:, EMAIN:].reshape(NW * ETAIL)

  agg = _get_agg(HID)

  mm = _mm_call(x, W_in)
  cnt = _get_cnt()(dst3, tdst)
  d2m, d1m, hp = _dis_call(cnt[0, :, :], cnt[1, :, :], mm)

  biases = [b_in] + [b_h[i] for i in range(NLAYER - 1)]
  for j in range(NLAYER):
    p = agg(hp, src1, dst3, tsrc, tdst)
    hp = _mid_call(p[0], p[1], hp, d2m, biases[j].reshape(1, HID), W_h[j])

  p = agg(hp, src1, dst3, tsrc, tdst)
  hp6 = _pre_call(p[0], p[1], hp, d2m, d1m, b_h[NLAYER - 1].reshape(1, HID))

  q = agg(hp6, src1, dst3, tsrc, tdst)
  wo = jnp.zeros((HID, HID), jnp.float32).at[:, :CLS].set(W_out)
  bo = jnp.zeros((1, HID), jnp.float32).at[0, :CLS].set(b_out)
  out128 = _out_call(q[0], q[1], hp6, d1m, wo, bo)
  return out128[:, :CLS]
